# Initial kernel scaffold; baseline (speedup 1.0000x reference)
import jax
import jax.numpy as jnp
from jax.experimental import pallas as pl


def kernel(inputs, embeddings):
    raise NotImplementedError("write your pallas kernel here")


# speedup vs baseline: 1.5312x; 1.5312x over previous
import functools

import jax
import jax.numpy as jnp
from jax import lax
from jax.experimental import pallas as pl
from jax.experimental.pallas import tpu as pltpu
from jax.experimental.pallas import tpu_sc as plsc

N_CODES = 1024
DIM = 64
N_ROWS = 32 * 1024
BLOCK_ROWS = 2048
COMMITMENT = 0.25

_SC_CORES = 2
_SC_SUBCORES = 16
_SC_WORKERS = _SC_CORES * _SC_SUBCORES
_ROWS_PER_WORKER = N_ROWS // _SC_WORKERS
_IDX_CHUNK = 128
_N_CHUNKS = _ROWS_PER_WORKER // _IDX_CHUNK


def _vq_body(x_ref, xsq_ref, esq_ref, e_ref, idx_ref, loss_ref):
    x = x_ref[...]
    e = e_ref[...]
    mm = lax.dot_general(x, e, (((1,), (1,)), ((), ())),
                         preferred_element_type=jnp.float32)
    d = (xsq_ref[...] + esq_ref[...]) - 2.0 * mm
    min_d = jnp.min(d, axis=1, keepdims=True)
    col = lax.broadcasted_iota(jnp.int32, d.shape, 1)
    idx = jnp.min(jnp.where(d == min_d, col, N_CODES), axis=1)
    idx_ref[...] = idx[:, None]

    @pl.when(pl.program_id(0) == 0)
    def _init():
        loss_ref[0, 0] = 0.0

    loss_ref[0, 0] += jnp.sum(min_d)

    @pl.when(pl.program_id(0) == pl.num_programs(0) - 1)
    def _finish():
        loss_ref[0, 0] *= (1.0 + COMMITMENT) / (N_ROWS * DIM)


def _distance_argmin(flat, x_sq, e_sq, embeddings):
    grid = N_ROWS // BLOCK_ROWS
    return pl.pallas_call(
        _vq_body,
        grid=(grid,),
        in_specs=[
            pl.BlockSpec((BLOCK_ROWS, DIM), lambda i: (i, 0)),
            pl.BlockSpec((BLOCK_ROWS, 1), lambda i: (i, 0)),
            pl.BlockSpec((1, N_CODES), lambda i: (0, 0)),
            pl.BlockSpec((N_CODES, DIM), lambda i: (0, 0)),
        ],
        out_specs=[
            pl.BlockSpec((BLOCK_ROWS, 1), lambda i: (i, 0)),
            pl.BlockSpec((1, 1), lambda i: (0, 0), memory_space=pltpu.SMEM),
        ],
        out_shape=[
            jax.ShapeDtypeStruct((N_ROWS, 1), jnp.int32),
            jax.ShapeDtypeStruct((1, 1), jnp.float32),
        ],
    )(flat, x_sq, e_sq, embeddings)


@functools.lru_cache(maxsize=None)
def _make_sc_gather():
    @functools.partial(
        pl.kernel,
        out_type=jax.ShapeDtypeStruct((N_ROWS, DIM), jnp.float32),
        mesh=plsc.VectorSubcoreMesh(core_axis_name="c", subcore_axis_name="s"),
        scratch_types=[
            pltpu.VMEM((_N_CHUNKS, _IDX_CHUNK), jnp.int32),
            pltpu.VMEM((_ROWS_PER_WORKER, DIM), jnp.float32),
            pltpu.SemaphoreType.DMA,
        ],
        compiler_params=pltpu.CompilerParams(use_tc_tiling_on_sc=False),
    )
    def _sc_gather(table_hbm, idx_hbm, out_hbm, idx_v, rows_v, sem):
        wid = lax.axis_index("s") * _SC_CORES + lax.axis_index("c")
        base = wid * _ROWS_PER_WORKER
        pltpu.sync_copy(idx_hbm.at[wid], idx_v)
        copies = [
            pltpu.async_copy(
                table_hbm.at[idx_v.at[j]],
                rows_v.at[pl.ds(j * _IDX_CHUNK, _IDX_CHUNK)],
                sem,
            )
            for j in range(_N_CHUNKS)
        ]
        for cp in copies:
            cp.wait()
        pltpu.sync_copy(rows_v, out_hbm.at[pl.ds(base, _ROWS_PER_WORKER)])

    return _sc_gather


def kernel(inputs, embeddings):
    flat = inputs.reshape(-1, DIM)
    x_sq = jnp.sum(flat ** 2, axis=1, keepdims=True)
    e_sq = jnp.sum(embeddings ** 2, axis=1)[None, :]
    idx2d, loss11 = _distance_argmin(flat, x_sq, e_sq, embeddings)
    idx3d = idx2d.reshape(_SC_WORKERS, _N_CHUNKS, _IDX_CHUNK)
    quantized = _make_sc_gather()(embeddings, idx3d)
    return (quantized.reshape(inputs.shape), loss11[0, 0], idx2d)
